# baseline (speedup 1.0000x reference)
import jax
import jax.numpy as jnp
from jax.experimental import pallas as pl
from jax.experimental.pallas import tpu as pltpu


def _decode_kernel(eid_ref, x_ref, w_ref, bias_ref, o_ref):
    e = eid_ref[pl.program_id(0)]

    acc = jax.lax.dot_general(
        x_ref[0], w_ref[e], (((1,), (0,)), ((), ())),
        precision=jax.lax.Precision.DEFAULT,
        preferred_element_type=jnp.float32)
    o_ref[0] = acc + bias_ref[e]


def kernel(x, eid, W, b):
    B, T, P = x.shape
    E, _, N = W.shape
    grid = (B, 2)
    grid_spec = pltpu.PrefetchScalarGridSpec(
        num_scalar_prefetch=1,
        grid=grid,
        in_specs=[
            pl.BlockSpec((1, T // 2, P), lambda bi, ti, se: (bi, ti, 0)),
            pl.BlockSpec((E, P, N), lambda bi, ti, se: (0, 0, 0),
                         pipeline_mode=pl.Buffered(buffer_count=1)),
            pl.BlockSpec((E, 1, N), lambda bi, ti, se: (0, 0, 0),
                         pipeline_mode=pl.Buffered(buffer_count=1)),
        ],
        out_specs=pl.BlockSpec((1, T // 2, N), lambda bi, ti, se: (bi, ti, 0)),
    )
    return pl.pallas_call(
        _decode_kernel,
        grid_spec=grid_spec,
        out_shape=jax.ShapeDtypeStruct((B, T, N), jnp.float32),
        compiler_params=pltpu.CompilerParams(
            dimension_semantics=("arbitrary", "arbitrary"),
            vmem_limit_bytes=100 * 1024 * 1024,
        ),
    )(eid, x, W, b.reshape(E, 1, N))


# speedup vs baseline: 1.0365x; 1.0365x over previous
import jax
import jax.numpy as jnp
from jax.experimental import pallas as pl
from jax.experimental.pallas import tpu as pltpu

_BT = 2048


def _decode_kernel(order_ref, seid_ref, x_ref, w_ref, bias_ref, o_ref):
    del order_ref, seid_ref
    acc = jax.lax.dot_general(
        x_ref[0], w_ref[0], (((1,), (0,)), ((), ())),
        precision=jax.lax.Precision.DEFAULT,
        preferred_element_type=jnp.float32)
    o_ref[0] = acc + bias_ref[0]


def kernel(x, eid, W, b):
    B, T, P = x.shape
    E, _, N = W.shape
    order = jnp.argsort(eid).astype(jnp.int32)
    seid = jnp.take(eid, order)
    grid = (B,)
    grid_spec = pltpu.PrefetchScalarGridSpec(
        num_scalar_prefetch=2,
        grid=grid,
        in_specs=[
            pl.BlockSpec((1, T, P), lambda bi, ordr, se: (ordr[bi], 0, 0)),
            pl.BlockSpec((1, P, N), lambda bi, ordr, se: (se[bi], 0, 0)),
            pl.BlockSpec((1, 1, N), lambda bi, ordr, se: (se[bi], 0, 0)),
        ],
        out_specs=pl.BlockSpec((1, T, N), lambda bi, ordr, se: (ordr[bi], 0, 0)),
    )
    return pl.pallas_call(
        _decode_kernel,
        grid_spec=grid_spec,
        out_shape=jax.ShapeDtypeStruct((B, T, N), jnp.float32),
        compiler_params=pltpu.CompilerParams(
            dimension_semantics=("arbitrary",),
        ),
    )(order, seid, x, W, b.reshape(E, 1, N))


# speedup vs baseline: 1.0506x; 1.0135x over previous
import jax
import jax.numpy as jnp
from jax.experimental import pallas as pl
from jax.experimental.pallas import tpu as pltpu

_BT = 2048


def _decode_kernel(order_ref, seid_ref, x_ref, w_ref, bias_ref, o_ref):
    del order_ref, seid_ref
    acc = jax.lax.dot_general(
        x_ref[0], w_ref[0], (((1,), (0,)), ((), ())),
        precision=jax.lax.Precision.DEFAULT,
        preferred_element_type=jnp.float32)
    o_ref[0] = acc + bias_ref[0]


def kernel(x, eid, W, b):
    B, T, P = x.shape
    E, _, N = W.shape
    iota = jax.lax.iota(jnp.int32, B)
    seid, order = jax.lax.sort((eid, iota), dimension=0, num_keys=1)
    grid = (B,)
    grid_spec = pltpu.PrefetchScalarGridSpec(
        num_scalar_prefetch=2,
        grid=grid,
        in_specs=[
            pl.BlockSpec((1, T, P), lambda bi, ordr, se: (ordr[bi], 0, 0)),
            pl.BlockSpec((1, P, N), lambda bi, ordr, se: (se[bi], 0, 0)),
            pl.BlockSpec((1, 1, N), lambda bi, ordr, se: (se[bi], 0, 0)),
        ],
        out_specs=pl.BlockSpec((1, T, N), lambda bi, ordr, se: (ordr[bi], 0, 0)),
    )
    return pl.pallas_call(
        _decode_kernel,
        grid_spec=grid_spec,
        out_shape=jax.ShapeDtypeStruct((B, T, N), jnp.float32),
        compiler_params=pltpu.CompilerParams(
            dimension_semantics=("arbitrary",),
        ),
    )(order, seid, x, W, b.reshape(E, 1, N))


# speedup vs baseline: 1.0511x; 1.0005x over previous
import jax
import jax.numpy as jnp
from jax.experimental import pallas as pl
from jax.experimental.pallas import tpu as pltpu


def _decode_kernel(order_ref, seid_ref, x_ref, w_ref, bias_ref, o_ref):
    del order_ref, seid_ref
    acc = jax.lax.dot_general(
        x_ref[0], w_ref[0], (((1,), (0,)), ((), ())),
        precision=jax.lax.Precision.DEFAULT,
        preferred_element_type=jnp.float32)
    o_ref[0] = acc + bias_ref[0]


def kernel(x, eid, W, b):
    B, T, P = x.shape
    E, _, N = W.shape
    iota = jax.lax.iota(jnp.int32, B)
    seid, order = jax.lax.sort((eid, iota), dimension=0, num_keys=1)
    grid = (B,)
    grid_spec = pltpu.PrefetchScalarGridSpec(
        num_scalar_prefetch=2,
        grid=grid,
        in_specs=[
            pl.BlockSpec((1, T, P), lambda bi, ordr, se: (ordr[bi], 0, 0)),
            pl.BlockSpec((1, P, N), lambda bi, ordr, se: (se[bi], 0, 0)),
            pl.BlockSpec((1, 1, N), lambda bi, ordr, se: (se[bi], 0, 0)),
        ],
        out_specs=pl.BlockSpec((1, T, N), lambda bi, ordr, se: (ordr[bi], 0, 0)),
    )
    return pl.pallas_call(
        _decode_kernel,
        grid_spec=grid_spec,
        out_shape=jax.ShapeDtypeStruct((B, T, N), jnp.float32),
        compiler_params=pltpu.CompilerParams(
            dimension_semantics=("arbitrary",),
        ),
    )(order, seid, x, W, b.reshape(E, 1, N))
